# Initial kernel scaffold; baseline (speedup 1.0000x reference)
import jax
import jax.numpy as jnp
from jax.experimental import pallas as pl


def kernel(x, metapath_embedding, edge_index, attn1_w, attn2):
    raise NotImplementedError("write your pallas kernel here")


# speedup vs baseline: 2.7821x; 2.7821x over previous
import jax
import jax.numpy as jnp
from jax.experimental import pallas as pl
from jax.experimental.pallas import tpu as pltpu

_N = 10000
_E = 160000
_HID = 128
_H = 4
_NB = 1024
_NP = 10240
_B = 640
_ECH = _E // _B


def _celu(v):
    return jnp.where(v > 0, v, 3.0 * (jnp.exp(v / 3.0) - 1.0))


def _body(x_ref, dst_ref, mpe_ref, w1_ref, w2_ref, out_ref, acc_ref, den_ref):
    nb = pl.program_id(0)
    ec = pl.program_id(1)

    @pl.when(ec == 0)
    def _zero():
        acc_ref[...] = jnp.zeros_like(acc_ref)
        den_ref[...] = jnp.zeros_like(den_ref)

    dst = dst_ref[0, :, :]
    node_ids = nb * _NB + jax.lax.broadcasted_iota(jnp.int32, (_NB, _B), 0)
    ohT = (node_ids == dst).astype(jnp.float32)

    a1_blk = _celu(jax.lax.dot_general(
        x_ref[...], w1_ref[...], (((1,), (1,)), ((), ())),
        preferred_element_type=jnp.float32))
    a1e = jax.lax.dot_general(
        ohT, a1_blk, (((0,), (0,)), ((), ())),
        preferred_element_type=jnp.float32)

    eft = _celu(mpe_ref[...])
    a2 = jax.lax.dot_general(
        eft, w2_ref[...], (((1,), (1,)), ((), ())),
        preferred_element_type=jnp.float32)
    ex = jnp.exp(_celu(a1e + a2))

    feat = (eft[:, None, :] * ex[:, :, None]).reshape(_B, _H * _HID)
    acc_ref[...] += jax.lax.dot_general(
        ohT, feat, (((1,), (0,)), ((), ())),
        preferred_element_type=jnp.float32)
    den_ref[...] += jax.lax.dot_general(
        ohT, ex, (((1,), (0,)), ((), ())),
        preferred_element_type=jnp.float32)

    @pl.when(ec == _ECH - 1)
    def _epilogue():
        den = den_ref[...]
        agg = acc_ref[...].reshape(_NB, _H, _HID)
        o = _celu(agg / (den[:, :, None] + 1e-9))
        o = jnp.where(den[:, :, None] > 0, o, 0.0)
        out_ref[...] = o.reshape(_NB, _H * _HID)


def kernel(x, metapath_embedding, edge_index, attn1_w, attn2):
    dst = edge_index[1].astype(jnp.int32).reshape(_ECH, 1, _B)
    xp = jnp.pad(x, ((0, _NP - _N), (0, 0)))
    w2 = attn2.reshape(_H, _HID)

    out = pl.pallas_call(
        _body,
        grid=(_NP // _NB, _ECH),
        in_specs=[
            pl.BlockSpec((_NB, _HID), lambda nb, ec: (nb, 0)),
            pl.BlockSpec((1, 1, _B), lambda nb, ec: (ec, 0, 0)),
            pl.BlockSpec((_B, _HID), lambda nb, ec: (ec, 0)),
            pl.BlockSpec((_H, _HID), lambda nb, ec: (0, 0)),
            pl.BlockSpec((_H, _HID), lambda nb, ec: (0, 0)),
        ],
        out_specs=pl.BlockSpec((_NB, _H * _HID), lambda nb, ec: (nb, 0)),
        out_shape=jax.ShapeDtypeStruct((_NP, _H * _HID), jnp.float32),
        scratch_shapes=[
            pltpu.VMEM((_NB, _H * _HID), jnp.float32),
            pltpu.VMEM((_NB, _H), jnp.float32),
        ],
    )(xp, dst, metapath_embedding, attn1_w, w2)
    return out[:_N]
